# Initial kernel scaffold; baseline (speedup 1.0000x reference)
import functools

import jax
import jax.numpy as jnp
from jax import lax
from jax.experimental import pallas as pl
from jax.experimental.pallas import tpu as pltpu
from jax.experimental.pallas import tpu_sc as plsc

N = 10000
E = 320000
D_IN = 128
D_E = 16
D_OUT = 128

NC = 2
NS = 16
NW = NC * NS
EPW = E // NW
K = 80
NCHUNK = EPW // K
RPT = N // NS

ZA_R = 25
ZB_R = 125


def _sc_body(src_hbm, dst_hbm, nf_hbm, ef_hbm,
             a_out, b_out, d_out,
             sidx, didx, rows, erows, ones_rows, zbuf_a, zbuf_b,
             a_sh, b_sh, d_sh, sem):
    c_id = lax.axis_index("c")
    s_id = lax.axis_index("s")
    wid = c_id * NS + s_id

    zvec = jnp.zeros((16,), jnp.float32)

    def fill_za(i, _):
        zbuf_a[i // 8, pl.ds((i % 8) * 16, 16)] = zvec
        return 0
    lax.fori_loop(0, ZA_R * 8, fill_za, 0)

    def fill_zb(i, _):
        zbuf_b[i, :] = zvec
        return 0
    lax.fori_loop(0, ZB_R, fill_zb, 0)

    onevec = jnp.where(lax.iota(jnp.int32, 16) == 0, 1.0, 0.0).astype(jnp.float32)

    def fill_ones(i, _):
        ones_rows[i, :] = onevec
        return 0
    lax.fori_loop(0, K, fill_ones, 0)

    row0 = s_id * RPT
    for j in range(RPT // ZA_R):
        pltpu.sync_copy(zbuf_a, a_sh.at[pl.ds(row0 + j * ZA_R, ZA_R)])
    for j in range(RPT // ZB_R):
        pltpu.sync_copy(zbuf_b, b_sh.at[pl.ds(row0 + j * ZB_R, ZB_R)])
        pltpu.sync_copy(zbuf_b, d_sh.at[pl.ds(row0 + j * ZB_R, ZB_R)])
    plsc.subcore_barrier()

    ebase = wid * EPW

    def chunk(i, _):
        base = ebase + i * K
        pltpu.sync_copy(src_hbm.at[pl.ds(base, K)], sidx)
        pltpu.sync_copy(dst_hbm.at[pl.ds(base, K)], didx)
        pltpu.sync_copy(ef_hbm.at[pl.ds(base, K)], erows)
        pltpu.async_copy(nf_hbm.at[sidx], rows, sem).wait()
        pltpu.sync_copy(rows, a_sh.at[didx], add=True)
        pltpu.sync_copy(erows, b_sh.at[didx], add=True)
        pltpu.sync_copy(ones_rows, d_sh.at[didx], add=True)
        return 0

    lax.fori_loop(0, NCHUNK, chunk, 0)
    plsc.subcore_barrier()

    pltpu.sync_copy(a_sh.at[pl.ds(row0, RPT)], a_out.at[c_id, pl.ds(row0, RPT)])
    pltpu.sync_copy(b_sh.at[pl.ds(row0, RPT)], b_out.at[c_id, pl.ds(row0, RPT)])
    pltpu.sync_copy(d_sh.at[pl.ds(row0, RPT)], d_out.at[c_id, pl.ds(row0, RPT)])


_sc_aggregate = functools.partial(
    pl.kernel,
    out_type=[
        jax.ShapeDtypeStruct((NC, N, D_IN), jnp.float32),
        jax.ShapeDtypeStruct((NC, N, D_E), jnp.float32),
        jax.ShapeDtypeStruct((NC, N, 16), jnp.float32),
    ],
    mesh=plsc.VectorSubcoreMesh(core_axis_name="c", subcore_axis_name="s"),
    scratch_types=[
        pltpu.VMEM((K,), jnp.int32),
        pltpu.VMEM((K,), jnp.int32),
        pltpu.VMEM((K, D_IN), jnp.float32),
        pltpu.VMEM((K, D_E), jnp.float32),
        pltpu.VMEM((K, 16), jnp.float32),
        pltpu.VMEM((ZA_R, D_IN), jnp.float32),
        pltpu.VMEM((ZB_R, 16), jnp.float32),
        pltpu.VMEM_SHARED((N, D_IN), jnp.float32),
        pltpu.VMEM_SHARED((N, D_E), jnp.float32),
        pltpu.VMEM_SHARED((N, 16), jnp.float32),
        pltpu.SemaphoreType.DMA,
    ],
)(_sc_body)


M_BLK = 2000


def _tc_body(a2, b2, d2, x, wm1t, wm2t, bm, wa1t, wa2t, ba, out):
    hp = jax.lax.Precision.HIGHEST
    a = a2[0] + a2[1]
    bv = b2[0] + b2[1]
    dcol = (d2[0] + d2[1])[:, 0:1]
    num = (jnp.dot(a, wm1t[...], precision=hp, preferred_element_type=jnp.float32)
           + jnp.dot(bv, wm2t[...], precision=hp, preferred_element_type=jnp.float32)
           + dcol * bm[...])
    h = num / jnp.maximum(dcol, 1.0)
    act = (jnp.dot(x[...], wa1t[...], precision=hp, preferred_element_type=jnp.float32)
           + jnp.dot(h, wa2t[...], precision=hp, preferred_element_type=jnp.float32)
           + ba[...])
    out[...] = jnp.maximum(act, 0.0)


def _tc_apply(a2, b2, d2, x, wm1t, wm2t, bm, wa1t, wa2t, ba):
    full = lambda s: pl.BlockSpec(s, lambda i: (0,) * len(s))
    return pl.pallas_call(
        _tc_body,
        grid=(N // M_BLK,),
        in_specs=[
            pl.BlockSpec((NC, M_BLK, D_IN), lambda i: (0, i, 0)),
            pl.BlockSpec((NC, M_BLK, D_E), lambda i: (0, i, 0)),
            pl.BlockSpec((NC, M_BLK, 16), lambda i: (0, i, 0)),
            pl.BlockSpec((M_BLK, D_IN), lambda i: (i, 0)),
            full((D_IN, D_OUT)),
            full((D_E, D_OUT)),
            full((1, D_OUT)),
            full((D_IN, D_OUT)),
            full((D_OUT, D_OUT)),
            full((1, D_OUT)),
        ],
        out_specs=pl.BlockSpec((M_BLK, D_OUT), lambda i: (i, 0)),
        out_shape=jax.ShapeDtypeStruct((N, D_OUT), jnp.float32),
    )(a2, b2, d2, x, wm1t, wm2t, bm, wa1t, wa2t, ba)


def kernel(nfeats, efeats, edge_index, W_msg, b_msg, W_apply, b_apply):
    nf = nfeats.reshape(N, D_IN)
    ef = efeats.reshape(E, D_E)
    src = edge_index[0]
    dst = edge_index[1]

    a2, b2, d2 = _sc_aggregate(src, dst, nf, ef)

    wm1t = W_msg[:, :D_IN].T
    wm2t = W_msg[:, D_IN:].T
    wa1t = W_apply[:, :D_IN].T
    wa2t = W_apply[:, D_IN:].T
    bm = b_msg.reshape(1, D_OUT)
    ba = b_apply.reshape(1, D_OUT)

    out = _tc_apply(a2, b2, d2, nf, wm1t, wm2t, bm, wa1t, wa2t, ba)
    return out.reshape(N, 1, D_OUT)


# speedup vs baseline: 3.1010x; 3.1010x over previous
import functools

import jax
import jax.numpy as jnp
from jax import lax
from jax.experimental import pallas as pl
from jax.experimental.pallas import tpu as pltpu
from jax.experimental.pallas import tpu_sc as plsc

N = 10000
E = 320000
D_IN = 128
D_E = 16
D_OUT = 128

NC = 2
NS = 16
NW = NC * NS
EPW = E // NW
EPT = E // NS
K = 80
NCHUNK_A = EPW // K
NCHUNK_S = EPT // K
NPAD = 10240
RPT = NPAD // NS


def _sc_a_body(src_hbm, dst_hbm, nf_hbm, a0_out, a1_out,
               sidx, didx, rows, a_sh, sem):
    c_id = lax.axis_index("c")
    s_id = lax.axis_index("s")
    wid = c_id * NS + s_id

    zvec = jnp.zeros((16,), jnp.float32)

    def fill_rows(i, _):
        rows[i // 8, pl.ds((i % 8) * 16, 16)] = zvec
        return 0
    lax.fori_loop(0, K * 8, fill_rows, 0)

    row0 = s_id * RPT
    for j in range(RPT // K):
        pltpu.sync_copy(rows, a_sh.at[pl.ds(row0 + j * K, K)])
    plsc.subcore_barrier()

    ebase = wid * EPW

    def chunk_a(i, _):
        base = ebase + i * K
        pltpu.sync_copy(src_hbm.at[pl.ds(base, K)], sidx)
        pltpu.sync_copy(dst_hbm.at[pl.ds(base, K)], didx)
        pltpu.async_copy(nf_hbm.at[sidx], rows, sem).wait()
        pltpu.sync_copy(rows, a_sh.at[didx], add=True)
        return 0

    lax.fori_loop(0, NCHUNK_A, chunk_a, 0)
    plsc.subcore_barrier()

    @pl.when(c_id == 0)
    def _():
        pltpu.sync_copy(a_sh.at[pl.ds(row0, RPT)], a0_out.at[pl.ds(row0, RPT)])

    @pl.when(c_id == 1)
    def _():
        pltpu.sync_copy(a_sh.at[pl.ds(row0, RPT)], a1_out.at[pl.ds(row0, RPT)])


_sc_a = functools.partial(
    pl.kernel,
    out_type=[
        jax.ShapeDtypeStruct((NPAD, D_IN), jnp.float32),
        jax.ShapeDtypeStruct((NPAD, D_IN), jnp.float32),
    ],
    mesh=plsc.VectorSubcoreMesh(core_axis_name="c", subcore_axis_name="s"),
    scratch_types=[
        pltpu.VMEM((K,), jnp.int32),
        pltpu.VMEM((K,), jnp.int32),
        pltpu.VMEM((K, D_IN), jnp.float32),
        pltpu.VMEM_SHARED((NPAD, D_IN), jnp.float32),
        pltpu.SemaphoreType.DMA,
    ],
)(_sc_a_body)


def _sc_bd_body(dst_hbm, ef_hbm, b_out, d_out,
                didx, erows, ones_rows, bd_sh, sem):
    c_id = lax.axis_index("c")
    s_id = lax.axis_index("s")

    zvec = jnp.zeros((16,), jnp.float32)

    def fill_erows(i, _):
        erows[i, :] = zvec
        return 0
    lax.fori_loop(0, K, fill_erows, 0)

    onevec = jnp.where(lax.iota(jnp.int32, 16) == 0, 1.0, 0.0).astype(jnp.float32)

    def fill_ones(i, _):
        ones_rows[i, :] = onevec
        return 0
    lax.fori_loop(0, K, fill_ones, 0)

    row0 = s_id * RPT
    for j in range(RPT // K):
        pltpu.sync_copy(erows, bd_sh.at[pl.ds(row0 + j * K, K)])
    plsc.subcore_barrier()

    ebase = s_id * EPT

    @pl.when(c_id == 0)
    def _():
        def chunk_b(i, _):
            base = ebase + i * K
            pltpu.sync_copy(dst_hbm.at[pl.ds(base, K)], didx)
            pltpu.sync_copy(ef_hbm.at[pl.ds(base, K)], erows)
            pltpu.sync_copy(erows, bd_sh.at[didx], add=True)
            return 0
        lax.fori_loop(0, NCHUNK_S, chunk_b, 0)

    @pl.when(c_id == 1)
    def _():
        def chunk_d(i, _):
            base = ebase + i * K
            pltpu.sync_copy(dst_hbm.at[pl.ds(base, K)], didx)
            pltpu.sync_copy(ones_rows, bd_sh.at[didx], add=True)
            return 0
        lax.fori_loop(0, NCHUNK_S, chunk_d, 0)

    plsc.subcore_barrier()

    @pl.when(c_id == 0)
    def _():
        pltpu.sync_copy(bd_sh.at[pl.ds(row0, RPT)], b_out.at[pl.ds(row0, RPT)])

    @pl.when(c_id == 1)
    def _():
        pltpu.sync_copy(bd_sh.at[pl.ds(row0, RPT)], d_out.at[pl.ds(row0, RPT)])


_sc_bd = functools.partial(
    pl.kernel,
    out_type=[
        jax.ShapeDtypeStruct((NPAD, D_E), jnp.float32),
        jax.ShapeDtypeStruct((NPAD, 16), jnp.float32),
    ],
    mesh=plsc.VectorSubcoreMesh(core_axis_name="c", subcore_axis_name="s"),
    scratch_types=[
        pltpu.VMEM((K,), jnp.int32),
        pltpu.VMEM((K, D_E), jnp.float32),
        pltpu.VMEM((K, 16), jnp.float32),
        pltpu.VMEM_SHARED((NPAD, 16), jnp.float32),
        pltpu.SemaphoreType.DMA,
    ],
)(_sc_bd_body)


M_BLK = 2000


def _tc_body(a0, a1, b, d, x, wm1t, wm2t, bm, wa1t, wa2t, ba, out):
    hp = jax.lax.Precision.HIGHEST
    a = a0[...] + a1[...]
    dcol = d[:, 0:1]
    num = (jnp.dot(a, wm1t[...], precision=hp, preferred_element_type=jnp.float32)
           + jnp.dot(b[...], wm2t[...], precision=hp, preferred_element_type=jnp.float32)
           + dcol * bm[...])
    h = num / jnp.maximum(dcol, 1.0)
    act = (jnp.dot(x[...], wa1t[...], precision=hp, preferred_element_type=jnp.float32)
           + jnp.dot(h, wa2t[...], precision=hp, preferred_element_type=jnp.float32)
           + ba[...])
    out[...] = jnp.maximum(act, 0.0)


def _tc_apply(a0, a1, b, d, x, wm1t, wm2t, bm, wa1t, wa2t, ba):
    full = lambda s: pl.BlockSpec(s, lambda i: (0,) * len(s))
    return pl.pallas_call(
        _tc_body,
        grid=(N // M_BLK,),
        in_specs=[
            pl.BlockSpec((M_BLK, D_IN), lambda i: (i, 0)),
            pl.BlockSpec((M_BLK, D_IN), lambda i: (i, 0)),
            pl.BlockSpec((M_BLK, D_E), lambda i: (i, 0)),
            pl.BlockSpec((M_BLK, 16), lambda i: (i, 0)),
            pl.BlockSpec((M_BLK, D_IN), lambda i: (i, 0)),
            full((D_IN, D_OUT)),
            full((D_E, D_OUT)),
            full((1, D_OUT)),
            full((D_IN, D_OUT)),
            full((D_OUT, D_OUT)),
            full((1, D_OUT)),
        ],
        out_specs=pl.BlockSpec((M_BLK, D_OUT), lambda i: (i, 0)),
        out_shape=jax.ShapeDtypeStruct((N, D_OUT), jnp.float32),
    )(a0, a1, b, d, x, wm1t, wm2t, bm, wa1t, wa2t, ba)


def kernel(nfeats, efeats, edge_index, W_msg, b_msg, W_apply, b_apply):
    nf = nfeats.reshape(N, D_IN)
    ef = efeats.reshape(E, D_E)
    src = edge_index[0]
    dst = edge_index[1]

    a0, a1 = _sc_a(src, dst, nf)
    b, d = _sc_bd(dst, ef)

    wm1t = W_msg[:, :D_IN].T
    wm2t = W_msg[:, D_IN:].T
    wa1t = W_apply[:, :D_IN].T
    wa2t = W_apply[:, D_IN:].T
    bm = b_msg.reshape(1, D_OUT)
    ba = b_apply.reshape(1, D_OUT)

    out = _tc_apply(a0, a1, b, d, nf, wm1t, wm2t, bm, wa1t, wa2t, ba)
    return out.reshape(N, 1, D_OUT)
